# baseline (speedup 1.0000x reference)
import functools

import jax
import jax.numpy as jnp
from jax import lax
from jax.experimental import pallas as pl
from jax.experimental.pallas import tpu as pltpu
from jax.experimental.pallas import tpu_sc as plsc

_S = 64.0
_M = 0.35
_STD = 0.0125
_BR = 16
_NC = 2
_NS = 16
_NW = _NC * _NS
_N = 1024
_BPW = _N // _NW


def _scale_body(x_ref, o_ref):
    o_ref[...] = x_ref[...] * _S


def _sc_fix_body(fidx_hbm, marg_hbm, out_hbm, idx_v, marg_v, val_v, sem):
    wid = lax.axis_index("s") * _NC + lax.axis_index("c")
    base = wid * _BPW
    pltpu.sync_copy(fidx_hbm.at[pl.ds(base, _BPW)], idx_v)
    pltpu.sync_copy(marg_hbm.at[pl.ds(base, _BPW)], marg_v)
    pltpu.async_copy(out_hbm.at[idx_v], val_v, sem).wait()
    for j in range(_BPW // 16):
        sl = pl.ds(j * 16, 16)
        val_v[sl] = val_v[sl] - marg_v[sl]
    pltpu.async_copy(val_v, out_hbm.at[idx_v], sem).wait()


_sc_fix = pl.kernel(
    _sc_fix_body,
    out_type=(),
    mesh=plsc.VectorSubcoreMesh(
        core_axis_name="c", subcore_axis_name="s", num_cores=_NC,
        num_subcores=_NS,
    ),
    scratch_types=[
        pltpu.VMEM((_BPW,), jnp.int32),
        pltpu.VMEM((_BPW,), jnp.float32),
        pltpu.VMEM((_BPW,), jnp.float32),
        pltpu.SemaphoreType.DMA,
    ],
)


def kernel(logits, label):
    n, c = logits.shape
    lab = label.astype(jnp.int32)
    marg = (jax.random.normal(jax.random.key(42), (n, 1), jnp.float32)
            * _STD + _M)[:, 0] * _S
    fidx = jnp.arange(n, dtype=jnp.int32) * c + lab
    out = pl.pallas_call(
        _scale_body,
        grid=(n // _BR,),
        in_specs=[pl.BlockSpec((_BR, c), lambda i: (i, 0))],
        out_specs=pl.BlockSpec((_BR, c), lambda i: (i, 0)),
        out_shape=jax.ShapeDtypeStruct((n, c), logits.dtype),
    )(logits)
    out_ref = jax.new_ref(out.reshape(-1))
    _sc_fix(fidx, marg, out_ref)
    return out_ref[...].reshape(n, c)


# speedup vs baseline: 2.0810x; 2.0810x over previous
import jax
import jax.numpy as jnp
from jax import lax
from jax.experimental import pallas as pl
from jax.experimental.pallas import tpu as pltpu
from jax.experimental.pallas import tpu_sc as plsc

_S = 64.0
_M = 0.35
_STD = 0.0125
_BR = 16
_NC = 2
_NS = 16
_NW = _NC * _NS
_N = 1024
_BPW = _N // _NW
_NG = _BPW // 8


def _scale_body(x_ref, o_ref):
    o_ref[...] = x_ref[...] * _S


def _sc_fix_body(lab_hbm, marg_hbm, out_ref, lab_v, marg_v, bufm_v, bufe_v,
                 sem):
    ncols = out_ref.shape[1]
    edge = (ncols // 128) * 128
    max_start = edge - 128
    wid = lax.axis_index("s") * _NC + lax.axis_index("c")
    base = wid * _BPW
    pltpu.sync_copy(lab_hbm.at[pl.ds(base, _BPW)], lab_v)
    pltpu.sync_copy(marg_hbm.at[pl.ds(base, _BPW)], marg_v)
    lab_c = [lab_v[pl.ds(k * 16, 16)] for k in range(_BPW // 16)]
    marg_c = [marg_v[pl.ds(k * 16, 16)] for k in range(_BPW // 16)]

    def col(j):
        return lab_c[j // 16][j % 16]

    def marg(j):
        return marg_c[j // 16][j % 16]

    lanes = lax.iota(jnp.int32, 16)

    def fix16(buf_slice, pos, m):
        width = buf_slice.shape[-1]
        chunk = jnp.clip(pos & ~15, 0, width - 16)
        sl = buf_slice.at[pl.ds(chunk, 16)]
        sl[...] = sl[...] - jnp.where(lanes == (pos - chunk), m, 0.0)

    for s in range(8):
        starts = []
        copies = []
        for g in range(_NG):
            j = 8 * g + s
            start = pl.multiple_of(
                jnp.minimum(col(j) & ~127, max_start), 128)
            starts.append(start)
            copies.append(pltpu.async_copy(
                out_ref.at[pl.ds(base + 8 * g, 8), pl.ds(start, 128)],
                bufm_v.at[g], sem))
            copies.append(pltpu.async_copy(
                out_ref.at[pl.ds(base + 8 * g, 8), pl.ds(edge, ncols - edge)],
                bufe_v.at[g], sem))
        for cp in copies:
            cp.wait()
        for g in range(_NG):
            j = 8 * g + s
            fix16(bufm_v.at[g, s], col(j) - starts[g], marg(j))
            fix16(bufe_v.at[g, s], col(j) - edge, marg(j))
        copies = []
        for g in range(_NG):
            copies.append(pltpu.async_copy(
                bufm_v.at[g],
                out_ref.at[pl.ds(base + 8 * g, 8), pl.ds(starts[g], 128)],
                sem))
            copies.append(pltpu.async_copy(
                bufe_v.at[g],
                out_ref.at[pl.ds(base + 8 * g, 8), pl.ds(edge, ncols - edge)],
                sem))
        for cp in copies:
            cp.wait()


_sc_fix = pl.kernel(
    _sc_fix_body,
    out_type=(),
    mesh=plsc.VectorSubcoreMesh(
        core_axis_name="c", subcore_axis_name="s", num_cores=_NC,
        num_subcores=_NS,
    ),
    scratch_types=[
        pltpu.VMEM((_BPW,), jnp.int32),
        pltpu.VMEM((_BPW,), jnp.float32),
        pltpu.VMEM((_NG, 8, 128), jnp.float32),
        pltpu.VMEM((_NG, 8, 32), jnp.float32),
        pltpu.SemaphoreType.DMA,
    ],
)


def kernel(logits, label):
    n, c = logits.shape
    lab = label.astype(jnp.int32)
    marg = (jax.random.normal(jax.random.key(42), (n, 1), jnp.float32)
            * _STD + _M)[:, 0] * _S
    out = pl.pallas_call(
        _scale_body,
        grid=(n // _BR,),
        in_specs=[pl.BlockSpec((_BR, c), lambda i: (i, 0))],
        out_specs=pl.BlockSpec((_BR, c), lambda i: (i, 0)),
        out_shape=jax.ShapeDtypeStruct((n, c), logits.dtype),
    )(logits)
    out_ref = jax.new_ref(out)
    _sc_fix(lab, marg, out_ref)
    return jax.freeze(out_ref)


# speedup vs baseline: 2.0833x; 1.0011x over previous
import jax
import jax.numpy as jnp
from jax import lax
from jax.experimental import pallas as pl
from jax.experimental.pallas import tpu as pltpu
from jax.experimental.pallas import tpu_sc as plsc

_S = 64.0
_M = 0.35
_STD = 0.0125
_BR = 32
_NC = 2
_NS = 16
_NW = _NC * _NS
_N = 1024
_BPW = _N // _NW
_NG = _BPW // 8


def _scale_body(x_ref, o_ref):
    o_ref[...] = x_ref[...] * _S


def _sc_fix_body(lab_hbm, marg_hbm, out_ref, lab_v, marg_v, bufm_v, bufe_v,
                 sem):
    ncols = out_ref.shape[1]
    edge = (ncols // 128) * 128
    max_start = edge - 128
    wid = lax.axis_index("s") * _NC + lax.axis_index("c")
    base = wid * _BPW
    pltpu.sync_copy(lab_hbm.at[pl.ds(base, _BPW)], lab_v)
    pltpu.sync_copy(marg_hbm.at[pl.ds(base, _BPW)], marg_v)
    lab_c = [lab_v[pl.ds(k * 16, 16)] for k in range(_BPW // 16)]
    marg_c = [marg_v[pl.ds(k * 16, 16)] for k in range(_BPW // 16)]

    def col(j):
        return lab_c[j // 16][j % 16]

    def marg(j):
        return marg_c[j // 16][j % 16]

    lanes = lax.iota(jnp.int32, 16)

    def fix16(buf_slice, pos, m):
        width = buf_slice.shape[-1]
        chunk = jnp.clip(pos & ~15, 0, width - 16)
        sl = buf_slice.at[pl.ds(chunk, 16)]
        sl[...] = sl[...] - jnp.where(lanes == (pos - chunk), m, 0.0)

    for s in range(8):
        starts = []
        copies = []
        for g in range(_NG):
            j = 8 * g + s
            start = pl.multiple_of(
                jnp.minimum(col(j) & ~127, max_start), 128)
            starts.append(start)
            copies.append(pltpu.async_copy(
                out_ref.at[pl.ds(base + 8 * g, 8), pl.ds(start, 128)],
                bufm_v.at[g], sem))
            copies.append(pltpu.async_copy(
                out_ref.at[pl.ds(base + 8 * g, 8), pl.ds(edge, ncols - edge)],
                bufe_v.at[g], sem))
        for cp in copies:
            cp.wait()
        for g in range(_NG):
            j = 8 * g + s
            fix16(bufm_v.at[g, s], col(j) - starts[g], marg(j))
            fix16(bufe_v.at[g, s], col(j) - edge, marg(j))
        copies = []
        for g in range(_NG):
            copies.append(pltpu.async_copy(
                bufm_v.at[g],
                out_ref.at[pl.ds(base + 8 * g, 8), pl.ds(starts[g], 128)],
                sem))
            copies.append(pltpu.async_copy(
                bufe_v.at[g],
                out_ref.at[pl.ds(base + 8 * g, 8), pl.ds(edge, ncols - edge)],
                sem))
        for cp in copies:
            cp.wait()


_sc_fix = pl.kernel(
    _sc_fix_body,
    out_type=(),
    mesh=plsc.VectorSubcoreMesh(
        core_axis_name="c", subcore_axis_name="s", num_cores=_NC,
        num_subcores=_NS,
    ),
    scratch_types=[
        pltpu.VMEM((_BPW,), jnp.int32),
        pltpu.VMEM((_BPW,), jnp.float32),
        pltpu.VMEM((_NG, 8, 128), jnp.float32),
        pltpu.VMEM((_NG, 8, 32), jnp.float32),
        pltpu.SemaphoreType.DMA,
    ],
)


def kernel(logits, label):
    n, c = logits.shape
    lab = label.astype(jnp.int32)
    marg = (jax.random.normal(jax.random.key(42), (n, 1), jnp.float32)
            * _STD + _M)[:, 0] * _S
    out = pl.pallas_call(
        _scale_body,
        grid=(n // _BR,),
        in_specs=[pl.BlockSpec((_BR, c), lambda i: (i, 0))],
        out_specs=pl.BlockSpec((_BR, c), lambda i: (i, 0)),
        out_shape=jax.ShapeDtypeStruct((n, c), logits.dtype),
    )(logits)
    out_ref = jax.new_ref(out)
    _sc_fix(lab, marg, out_ref)
    return jax.freeze(out_ref)
